# Initial kernel scaffold; baseline (speedup 1.0000x reference)
import jax
import jax.numpy as jnp
from jax.experimental import pallas as pl


def kernel(x, edge_index, W1, a_src1, a_dst1, b1, W2, a_src2, a_dst2, b2, Wc, bc):
    raise NotImplementedError("write your pallas kernel here")


# speedup vs baseline: 55.5753x; 55.5753x over previous
import functools

import jax
import jax.numpy as jnp
from jax import lax
from jax.experimental import pallas as pl
from jax.experimental.pallas import tpu as pltpu
from jax.experimental.pallas import tpu_sc as plsc

N_NODES = 10000
N_EDGES = 320000
NC = 2
NS = 16
NW = NC * NS
E_PER_W = N_EDGES // NW
CHUNK = 80
NCHUNK = E_PER_W // CHUNK
NPAD = 10240
ROWS_PER_TILE = NPAD // NS
D = 16


def _tc_layer1(x_ref, w_ref, as_ref, ad_ref, h_ref, asn_ref, adn_ref):
    h = jnp.dot(x_ref[...], w_ref[...], preferred_element_type=jnp.float32)
    h_ref[...] = h
    asn_ref[...] = (h * as_ref[...]).sum(axis=1)
    adn_ref[...] = (h * ad_ref[...]).sum(axis=1)


def _tc_layer2(acc_ref, den_ref, b_ref, w_ref, as_ref, ad_ref,
               h_ref, asn_ref, adn_ref):
    acc = acc_ref[0] + acc_ref[1]
    den = den_ref[0] + den_ref[1]
    h1 = acc[:N_NODES] / (den[:N_NODES, None] + 1e-16) + b_ref[...]
    x2 = jnp.maximum(h1, 0.0)
    h2 = jnp.dot(x2, w_ref[...], preferred_element_type=jnp.float32)
    h_ref[...] = h2
    asn_ref[...] = (h2 * as_ref[...]).sum(axis=1)
    adn_ref[...] = (h2 * ad_ref[...]).sum(axis=1)


def _tc_head(acc_ref, den_ref, b_ref, wc_ref, bc_ref, h_ref, s_ref):
    acc = acc_ref[0] + acc_ref[1]
    den = den_ref[0] + den_ref[1]
    h2 = acc[:N_NODES, :8] / (den[:N_NODES, None] + 1e-16) + b_ref[...]
    h2 = jnp.maximum(h2, 0.0)
    h_ref[...] = h2
    logits = jnp.dot(h2, wc_ref[...], preferred_element_type=jnp.float32)
    s_ref[...] = jax.nn.sigmoid(logits + bc_ref[...])


def _build_sc_edge():
    mesh = plsc.VectorSubcoreMesh(core_axis_name="c", subcore_axis_name="s")
    return functools.partial(
        pl.kernel,
        out_type=(
            jax.ShapeDtypeStruct((NC, NPAD, D), jnp.float32),
            jax.ShapeDtypeStruct((NC, NPAD), jnp.float32),
        ),
        mesh=mesh,
        compiler_params=pltpu.CompilerParams(
            use_tc_tiling_on_sc=False, needs_layout_passes=False),
        scratch_types=[
        pltpu.VMEM((NCHUNK, CHUNK), jnp.int32),
        pltpu.VMEM((NCHUNK, CHUNK), jnp.int32),
        pltpu.VMEM((N_NODES,), jnp.float32),
        pltpu.VMEM((N_NODES,), jnp.float32),
        pltpu.VMEM((NCHUNK, CHUNK), jnp.float32),
        pltpu.VMEM((CHUNK, D), jnp.float32),
        pltpu.VMEM((ROWS_PER_TILE, D), jnp.float32),
        pltpu.VMEM((ROWS_PER_TILE,), jnp.float32),
            pltpu.VMEM_SHARED((NPAD, D), jnp.float32),
            pltpu.VMEM_SHARED((NPAD,), jnp.float32),
            pltpu.SemaphoreType.DMA,
        ],
    )(_sc_edge_body)


def _sc_edge_body(src_hbm, dst_hbm, asn_hbm, adn_hbm, h_hbm, acc_out, den_out,
             src_v, dst_v, as_t, ad_t, ex_v, rows_v, zrows, zden,
             acc_sh, den_sh, sem):
    c = lax.axis_index("c")
    s = lax.axis_index("s")
    wid = c * NS + s

    pltpu.sync_copy(src_hbm.at[wid], src_v)
    pltpu.sync_copy(dst_hbm.at[wid], dst_v)
    pltpu.sync_copy(asn_hbm, as_t)
    pltpu.sync_copy(adn_hbm, ad_t)

    zero16 = jnp.zeros((16,), jnp.float32)

    def zrow_body(r, carry):
        zrows[r, :] = zero16
        return carry

    lax.fori_loop(0, ROWS_PER_TILE, zrow_body, 0)

    def zden_body(r, carry):
        zden[pl.ds(r * 16, 16)] = zero16
        return carry

    lax.fori_loop(0, ROWS_PER_TILE // 16, zden_body, 0)

    row0 = s * ROWS_PER_TILE
    pltpu.sync_copy(zrows, acc_sh.at[pl.ds(row0, ROWS_PER_TILE)])
    pltpu.sync_copy(zden, den_sh.at[pl.ds(row0, ROWS_PER_TILE)])
    plsc.subcore_barrier()

    def chunk_body(ci, carry):
        cp = pltpu.async_copy(h_hbm.at[src_v.at[ci]], rows_v, sem)

        def grp_body(g, inner):
            sidx = src_v[ci, pl.ds(g * 16, 16)]
            didx = dst_v[ci, pl.ds(g * 16, 16)]
            e = plsc.load_gather(as_t, [sidx]) + plsc.load_gather(ad_t, [didx])
            e = jnp.where(e >= 0.0, e, 0.2 * e)
            ex_v[ci, pl.ds(g * 16, 16)] = jnp.exp(e)
            return inner

        lax.fori_loop(0, CHUNK // 16, grp_body, 0)
        cp.wait()

        def mul_body(g, inner):
            exv = ex_v[ci, pl.ds(g * 16, 16)]
            base = g * 16
            for j2 in range(16):
                rows_v[base + j2, :] = rows_v[base + j2, :] * exv[j2]
            return inner

        lax.fori_loop(0, CHUNK // 16, mul_body, 0)

        pltpu.sync_copy(rows_v, acc_sh.at[dst_v.at[ci]], add=True)
        pltpu.sync_copy(ex_v.at[ci], den_sh.at[dst_v.at[ci]], add=True)
        return carry

    lax.fori_loop(0, NCHUNK, chunk_body, 0)
    plsc.subcore_barrier()

    pltpu.sync_copy(acc_sh.at[pl.ds(row0, ROWS_PER_TILE)],
                    acc_out.at[c, pl.ds(row0, ROWS_PER_TILE)])
    pltpu.sync_copy(den_sh.at[pl.ds(row0, ROWS_PER_TILE)],
                    den_out.at[c, pl.ds(row0, ROWS_PER_TILE)])


_SC_EDGE_CACHE = []


def _sc_edge(src, dst, asn, adn, h):
    if not _SC_EDGE_CACHE:
        _SC_EDGE_CACHE.append(_build_sc_edge())
    return _SC_EDGE_CACHE[0](src, dst, asn, adn, h)


def kernel(x, edge_index, W1, a_src1, a_dst1, b1, W2, a_src2, a_dst2, b2,
           Wc, bc):
    src = edge_index[0].astype(jnp.int32).reshape(NW, NCHUNK, CHUNK)
    dst = edge_index[1].astype(jnp.int32).reshape(NW, NCHUNK, CHUNK)

    h1, as1, ad1 = pl.pallas_call(
        _tc_layer1,
        out_shape=(
            jax.ShapeDtypeStruct((N_NODES, D), jnp.float32),
            jax.ShapeDtypeStruct((N_NODES,), jnp.float32),
            jax.ShapeDtypeStruct((N_NODES,), jnp.float32),
        ),
    )(x, W1, a_src1.reshape(1, D), a_dst1.reshape(1, D))

    acc1, den1 = _sc_edge(src, dst, as1, ad1, h1)

    W2p = jnp.concatenate([W2, jnp.zeros((16, 8), jnp.float32)], axis=1)
    a2sp = jnp.concatenate([a_src2, jnp.zeros((8,), jnp.float32)])
    a2dp = jnp.concatenate([a_dst2, jnp.zeros((8,), jnp.float32)])

    h2, as2, ad2 = pl.pallas_call(
        _tc_layer2,
        out_shape=(
            jax.ShapeDtypeStruct((N_NODES, D), jnp.float32),
            jax.ShapeDtypeStruct((N_NODES,), jnp.float32),
            jax.ShapeDtypeStruct((N_NODES,), jnp.float32),
        ),
    )(acc1, den1, b1.reshape(1, D), W2p, a2sp.reshape(1, D),
      a2dp.reshape(1, D))

    acc2, den2 = _sc_edge(src, dst, as2, ad2, h2)

    h_out, scores = pl.pallas_call(
        _tc_head,
        out_shape=(
            jax.ShapeDtypeStruct((N_NODES, 8), jnp.float32),
            jax.ShapeDtypeStruct((N_NODES, 1), jnp.float32),
        ),
    )(acc2, den2, b2.reshape(1, 8), Wc, bc.reshape(1, 1))

    return (h_out, scores)


# speedup vs baseline: 77.4997x; 1.3945x over previous
import functools

import jax
import jax.numpy as jnp
from jax import lax
from jax.experimental import pallas as pl
from jax.experimental.pallas import tpu as pltpu
from jax.experimental.pallas import tpu_sc as plsc

N_NODES = 10000
N_EDGES = 320000
NC = 2
NS = 16
NW = NC * NS
CHUNK = 128
NCHUNK = 80
E_PAD = NW * NCHUNK * CHUNK
NPAD = 10240
ROWS_PER_TILE = NPAD // NS
D = 16


def _tc_layer1(x_ref, w_ref, as_ref, ad_ref, h_ref, asn_ref, adn_ref):
    h = jnp.dot(x_ref[...], w_ref[...], preferred_element_type=jnp.float32)
    h_ref[...] = h
    asn_ref[...] = (h * as_ref[...]).sum(axis=1)
    adn_ref[...] = (h * ad_ref[...]).sum(axis=1)


def _tc_layer2(acc_ref, den_ref, b_ref, w_ref, as_ref, ad_ref,
               h_ref, asn_ref, adn_ref):
    acc = acc_ref[0] + acc_ref[1]
    den = den_ref[0] + den_ref[1]
    h1 = acc[:N_NODES] / (den[:N_NODES, None] + 1e-16) + b_ref[...]
    x2 = jnp.maximum(h1, 0.0)
    h2 = jnp.dot(x2, w_ref[...], preferred_element_type=jnp.float32)
    col = lax.broadcasted_iota(jnp.int32, (N_NODES, D), 1)
    h_ref[...] = jnp.where(col == 8, 1.0, h2)
    asn_ref[...] = (h2 * as_ref[...]).sum(axis=1)
    adn_ref[...] = (h2 * ad_ref[...]).sum(axis=1)


def _tc_head(acc_ref, b_ref, wc_ref, bc_ref, h_ref, s_ref):
    acc = acc_ref[0] + acc_ref[1]
    den = acc[:N_NODES, 8]
    h2 = acc[:N_NODES, :8] / (den[:, None] + 1e-16) + b_ref[...]
    h2 = jnp.maximum(h2, 0.0)
    h_ref[...] = h2
    logits = jnp.dot(h2, wc_ref[...], preferred_element_type=jnp.float32)
    s_ref[...] = jax.nn.sigmoid(logits + bc_ref[...])


def _sc_edge_body(with_den, src_hbm, dst_hbm, asn_hbm, adn_hbm, h_hbm,
                  *refs):
    if with_den:
        (acc_out, den_out, src_v, dst_v, as_t, ad_t, ex_v, rows0, rows1,
         zrows, zden, acc_sh, den_sh, gsem0, gsem1) = refs
    else:
        (acc_out, src_v, dst_v, as_t, ad_t, ex_v, rows0, rows1,
         zrows, acc_sh, gsem0, gsem1) = refs
    c = lax.axis_index("c")
    s = lax.axis_index("s")
    wid = c * NS + s

    pltpu.sync_copy(src_hbm.at[wid], src_v)
    pltpu.sync_copy(dst_hbm.at[wid], dst_v)
    pltpu.sync_copy(asn_hbm, as_t)
    pltpu.sync_copy(adn_hbm, ad_t)

    zero16 = jnp.zeros((16,), jnp.float32)

    def zrow_body(r, carry):
        zrows[r, :] = zero16
        return carry

    lax.fori_loop(0, ROWS_PER_TILE, zrow_body, 0)

    row0 = s * ROWS_PER_TILE
    pltpu.sync_copy(zrows, acc_sh.at[pl.ds(row0, ROWS_PER_TILE)])
    if with_den:
        def zden_body(r, carry):
            zden[pl.ds(r * 16, 16)] = zero16
            return carry

        lax.fori_loop(0, ROWS_PER_TILE // 16, zden_body, 0)
        pltpu.sync_copy(zden, den_sh.at[pl.ds(row0, ROWS_PER_TILE)])
    plsc.subcore_barrier()

    def compute_ex(ci):
        def grp_body(g, inner):
            sidx = src_v[ci, pl.ds(g * 16, 16)]
            didx = dst_v[ci, pl.ds(g * 16, 16)]
            e = plsc.load_gather(as_t, [sidx]) + plsc.load_gather(ad_t, [didx])
            e = jnp.where(e >= 0.0, e, 0.2 * e)
            ex_v[ci, pl.ds(g * 16, 16)] = jnp.exp(e)
            return inner

        lax.fori_loop(0, CHUNK // 16, grp_body, 0)

    def process(ci, rows_v):
        def mul_body(g, inner):
            exv = ex_v[ci, pl.ds(g * 16, 16)]
            base = g * 16
            for j2 in range(16):
                rows_v[base + j2, :] = rows_v[base + j2, :] * exv[j2]
            return inner

        lax.fori_loop(0, CHUNK // 16, mul_body, 0)
        pltpu.sync_copy(rows_v, acc_sh.at[dst_v.at[ci]], add=True)
        if with_den:
            pltpu.sync_copy(ex_v.at[ci], den_sh.at[dst_v.at[ci]], add=True)

    def gather(ci, rows_v, sem):
        return pltpu.async_copy(h_hbm.at[src_v.at[ci]], rows_v, sem)

    gather(0, rows0, gsem0)
    npair = NCHUNK // 2

    def pair_body(p, carry):
        ci = 2 * p
        gather(ci + 1, rows1, gsem1)
        compute_ex(ci)
        pltpu.make_async_copy(h_hbm.at[src_v.at[0]], rows0, gsem0).wait()
        process(ci, rows0)

        @pl.when(p < npair - 1)
        def _():
            gather(ci + 2, rows0, gsem0)

        compute_ex(ci + 1)
        pltpu.make_async_copy(h_hbm.at[src_v.at[0]], rows1, gsem1).wait()
        process(ci + 1, rows1)
        return carry

    lax.fori_loop(0, npair, pair_body, 0)
    plsc.subcore_barrier()

    pltpu.sync_copy(acc_sh.at[pl.ds(row0, ROWS_PER_TILE)],
                    acc_out.at[c, pl.ds(row0, ROWS_PER_TILE)])
    if with_den:
        pltpu.sync_copy(den_sh.at[pl.ds(row0, ROWS_PER_TILE)],
                        den_out.at[c, pl.ds(row0, ROWS_PER_TILE)])


def _build_sc_edge(with_den):
    mesh = plsc.VectorSubcoreMesh(core_axis_name="c", subcore_axis_name="s")
    acc_t = jax.ShapeDtypeStruct((NC, NPAD, D), jnp.float32)
    den_t = jax.ShapeDtypeStruct((NC, NPAD), jnp.float32)
    scratch = [
        pltpu.VMEM((NCHUNK, CHUNK), jnp.int32),
        pltpu.VMEM((NCHUNK, CHUNK), jnp.int32),
        pltpu.VMEM((NPAD,), jnp.float32),
        pltpu.VMEM((NPAD,), jnp.float32),
        pltpu.VMEM((NCHUNK, CHUNK), jnp.float32),
        pltpu.VMEM((CHUNK, D), jnp.float32),
        pltpu.VMEM((CHUNK, D), jnp.float32),
        pltpu.VMEM((ROWS_PER_TILE, D), jnp.float32),
    ]
    if with_den:
        scratch.append(pltpu.VMEM((ROWS_PER_TILE,), jnp.float32))
    scratch.append(pltpu.VMEM_SHARED((NPAD, D), jnp.float32))
    if with_den:
        scratch.append(pltpu.VMEM_SHARED((NPAD,), jnp.float32))
    scratch += [pltpu.SemaphoreType.DMA, pltpu.SemaphoreType.DMA]
    return pl.kernel(
        functools.partial(_sc_edge_body, with_den),
        out_type=(acc_t, den_t) if with_den else acc_t,
        mesh=mesh,
        compiler_params=pltpu.CompilerParams(
            use_tc_tiling_on_sc=False, needs_layout_passes=False),
        scratch_types=scratch,
    )


_SC_EDGE_CACHE = {}


def _sc_edge(with_den, *args):
    if with_den not in _SC_EDGE_CACHE:
        _SC_EDGE_CACHE[with_den] = _build_sc_edge(with_den)
    return _SC_EDGE_CACHE[with_den](*args)


def kernel(x, edge_index, W1, a_src1, a_dst1, b1, W2, a_src2, a_dst2, b2,
           Wc, bc):
    npad_e = E_PAD - N_EDGES
    src = jnp.concatenate(
        [edge_index[0].astype(jnp.int32), jnp.zeros((npad_e,), jnp.int32)]
    ).reshape(NW, NCHUNK, CHUNK)
    dst = jnp.concatenate(
        [edge_index[1].astype(jnp.int32),
         jnp.full((npad_e,), N_NODES, jnp.int32)]
    ).reshape(NW, NCHUNK, CHUNK)

    zpad = jnp.zeros((NPAD - N_NODES,), jnp.float32)

    h1, as1, ad1 = pl.pallas_call(
        _tc_layer1,
        out_shape=(
            jax.ShapeDtypeStruct((N_NODES, D), jnp.float32),
            jax.ShapeDtypeStruct((N_NODES,), jnp.float32),
            jax.ShapeDtypeStruct((N_NODES,), jnp.float32),
        ),
    )(x, W1, a_src1.reshape(1, D), a_dst1.reshape(1, D))

    acc1, den1 = _sc_edge(True, src, dst,
                          jnp.concatenate([as1, zpad]),
                          jnp.concatenate([ad1, zpad]), h1)

    W2p = jnp.concatenate([W2, jnp.zeros((16, 8), jnp.float32)], axis=1)
    a2sp = jnp.concatenate([a_src2, jnp.zeros((8,), jnp.float32)])
    a2dp = jnp.concatenate([a_dst2, jnp.zeros((8,), jnp.float32)])

    h2, as2, ad2 = pl.pallas_call(
        _tc_layer2,
        out_shape=(
            jax.ShapeDtypeStruct((N_NODES, D), jnp.float32),
            jax.ShapeDtypeStruct((N_NODES,), jnp.float32),
            jax.ShapeDtypeStruct((N_NODES,), jnp.float32),
        ),
    )(acc1, den1, b1.reshape(1, D), W2p, a2sp.reshape(1, D),
      a2dp.reshape(1, D))

    acc2 = _sc_edge(False, src, dst,
                    jnp.concatenate([as2, zpad]),
                    jnp.concatenate([ad2, zpad]), h2)

    h_out, scores = pl.pallas_call(
        _tc_head,
        out_shape=(
            jax.ShapeDtypeStruct((N_NODES, 8), jnp.float32),
            jax.ShapeDtypeStruct((N_NODES, 1), jnp.float32),
        ),
    )(acc2, b2.reshape(1, 8), Wc, bc.reshape(1, 1))

    return (h_out, scores)


# speedup vs baseline: 78.4565x; 1.0123x over previous
import functools

import jax
import jax.numpy as jnp
from jax import lax
from jax.experimental import pallas as pl
from jax.experimental.pallas import tpu as pltpu
from jax.experimental.pallas import tpu_sc as plsc

N_NODES = 10000
N_EDGES = 320000
NC = 2
NS = 16
NW = NC * NS
CHUNK = 128
NCHUNK = 80
E_PAD = NW * NCHUNK * CHUNK
NPAD = 10240
ROWS_PER_TILE = NPAD // NS
D = 16


def _tc_layer1(x_ref, w_ref, as_ref, ad_ref, h_ref, asn_ref, adn_ref):
    h = jnp.dot(x_ref[...], w_ref[...], preferred_element_type=jnp.float32)
    h_ref[...] = h
    asn_ref[...] = (h * as_ref[...]).sum(axis=1)
    adn_ref[...] = (h * ad_ref[...]).sum(axis=1)


def _tc_layer2(acc_ref, den_ref, b_ref, w_ref, as_ref, ad_ref,
               h_ref, asn_ref, adn_ref):
    acc = acc_ref[0] + acc_ref[1]
    den = den_ref[0] + den_ref[1]
    h1 = acc[:N_NODES] / (den[:N_NODES, None] + 1e-16) + b_ref[...]
    x2 = jnp.maximum(h1, 0.0)
    h2 = jnp.dot(x2, w_ref[...], preferred_element_type=jnp.float32)
    col = lax.broadcasted_iota(jnp.int32, (N_NODES, D), 1)
    h_ref[...] = jnp.where(col == 8, 1.0, h2)
    asn_ref[...] = (h2 * as_ref[...]).sum(axis=1)
    adn_ref[...] = (h2 * ad_ref[...]).sum(axis=1)


def _tc_head(acc_ref, b_ref, wc_ref, bc_ref, h_ref, s_ref):
    acc = acc_ref[0] + acc_ref[1]
    den = acc[:N_NODES, 8]
    h2 = acc[:N_NODES, :8] / (den[:, None] + 1e-16) + b_ref[...]
    h2 = jnp.maximum(h2, 0.0)
    h_ref[...] = h2
    logits = jnp.dot(h2, wc_ref[...], preferred_element_type=jnp.float32)
    s_ref[...] = jax.nn.sigmoid(logits + bc_ref[...])


def _sc_edge_body(with_den, src_hbm, dst_hbm, asn_hbm, adn_hbm, h_hbm,
                  *refs):
    if with_den:
        (acc_out, den_out, src_v, dst_v, as_t, ad_t, ex_v,
         rows0, rows1, rows2, rows3, zrows, zden, acc_sh, den_sh,
         gsem0, gsem1, gsem2, gsem3, ssem0, ssem1, ssem2, ssem3,
         dsem0, dsem1, dsem2, dsem3) = refs
    else:
        (acc_out, src_v, dst_v, as_t, ad_t, ex_v,
         rows0, rows1, rows2, rows3, zrows, acc_sh,
         gsem0, gsem1, gsem2, gsem3, ssem0, ssem1, ssem2, ssem3) = refs
        dsem0 = dsem1 = dsem2 = dsem3 = None
    c = lax.axis_index("c")
    s = lax.axis_index("s")
    wid = c * NS + s

    pltpu.sync_copy(src_hbm.at[wid], src_v)
    pltpu.sync_copy(dst_hbm.at[wid], dst_v)
    pltpu.sync_copy(asn_hbm, as_t)
    pltpu.sync_copy(adn_hbm, ad_t)

    zero16 = jnp.zeros((16,), jnp.float32)

    def zrow_body(r, carry):
        zrows[r, :] = zero16
        return carry

    lax.fori_loop(0, ROWS_PER_TILE, zrow_body, 0)

    row0 = s * ROWS_PER_TILE
    pltpu.sync_copy(zrows, acc_sh.at[pl.ds(row0, ROWS_PER_TILE)])
    if with_den:
        def zden_body(r, carry):
            zden[pl.ds(r * 16, 16)] = zero16
            return carry

        lax.fori_loop(0, ROWS_PER_TILE // 16, zden_body, 0)
        pltpu.sync_copy(zden, den_sh.at[pl.ds(row0, ROWS_PER_TILE)])
    plsc.subcore_barrier()

    def compute_ex(ci):
        def grp_body(g, inner):
            sidx = src_v[ci, pl.ds(g * 16, 16)]
            didx = dst_v[ci, pl.ds(g * 16, 16)]
            e = plsc.load_gather(as_t, [sidx]) + plsc.load_gather(ad_t, [didx])
            e = jnp.where(e >= 0.0, e, 0.2 * e)
            ex_v[ci, pl.ds(g * 16, 16)] = jnp.exp(e)
            return inner

        lax.fori_loop(0, CHUNK // 16, grp_body, 0)

    def gather(ci, rows_v, sem):
        return pltpu.async_copy(h_hbm.at[src_v.at[ci]], rows_v, sem)

    def wait_gather(rows_v, sem):
        pltpu.make_async_copy(h_hbm.at[src_v.at[0]], rows_v, sem).wait()

    def scatter(ci, rows_v, ssem, dsem):
        pltpu.async_copy(rows_v, acc_sh.at[dst_v.at[ci]], ssem, add=True)
        if with_den:
            pltpu.async_copy(ex_v.at[ci], den_sh.at[dst_v.at[ci]], dsem,
                             add=True)

    def wait_scatter(rows_v, ssem, dsem):
        pltpu.make_async_copy(rows_v, acc_sh.at[dst_v.at[0]], ssem).wait()
        if with_den:
            pltpu.make_async_copy(ex_v.at[0], den_sh.at[dst_v.at[0]],
                                  dsem).wait()

    rows = (rows0, rows1, rows2, rows3)
    gsem = (gsem0, gsem1, gsem2, gsem3)
    ssem = (ssem0, ssem1, ssem2, ssem3)
    dsem = (dsem0, dsem1, dsem2, dsem3)
    nquad = NCHUNK // 4
    gather(0, rows[0], gsem[0])
    gather(1, rows[1], gsem[1])

    def quad_body(q, carry):
        ci = 4 * q
        for k in range(4):
            j = ci + k
            compute_ex(j)
            wait_gather(rows[k], gsem[k])

            def mul_body(g, inner):
                exv = ex_v[j, pl.ds(g * 16, 16)]
                base = g * 16
                for j2 in range(16):
                    rows[k][base + j2, :] = rows[k][base + j2, :] * exv[j2]
                return inner

            lax.fori_loop(0, CHUNK // 16, mul_body, 0)
            scatter(j, rows[k], ssem[k], dsem[k])

            kb = (k + 2) % 4
            if k < 2:
                @pl.when(q > 0)
                def _():
                    wait_scatter(rows[kb], ssem[kb], dsem[kb])
                gather(j + 2, rows[kb], gsem[kb])
            else:
                @pl.when(q < nquad - 1)
                def _():
                    wait_scatter(rows[kb], ssem[kb], dsem[kb])
                    gather(j + 2, rows[kb], gsem[kb])
        return carry

    lax.fori_loop(0, nquad, quad_body, 0)
    for k in range(4):
        wait_scatter(rows[k], ssem[k], dsem[k])
    plsc.subcore_barrier()

    pltpu.sync_copy(acc_sh.at[pl.ds(row0, ROWS_PER_TILE)],
                    acc_out.at[c, pl.ds(row0, ROWS_PER_TILE)])
    if with_den:
        pltpu.sync_copy(den_sh.at[pl.ds(row0, ROWS_PER_TILE)],
                        den_out.at[c, pl.ds(row0, ROWS_PER_TILE)])


def _build_sc_edge(with_den):
    mesh = plsc.VectorSubcoreMesh(core_axis_name="c", subcore_axis_name="s")
    acc_t = jax.ShapeDtypeStruct((NC, NPAD, D), jnp.float32)
    den_t = jax.ShapeDtypeStruct((NC, NPAD), jnp.float32)
    scratch = [
        pltpu.VMEM((NCHUNK, CHUNK), jnp.int32),
        pltpu.VMEM((NCHUNK, CHUNK), jnp.int32),
        pltpu.VMEM((NPAD,), jnp.float32),
        pltpu.VMEM((NPAD,), jnp.float32),
        pltpu.VMEM((NCHUNK, CHUNK), jnp.float32),
        pltpu.VMEM((CHUNK, D), jnp.float32),
        pltpu.VMEM((CHUNK, D), jnp.float32),
        pltpu.VMEM((CHUNK, D), jnp.float32),
        pltpu.VMEM((CHUNK, D), jnp.float32),
        pltpu.VMEM((ROWS_PER_TILE, D), jnp.float32),
    ]
    if with_den:
        scratch.append(pltpu.VMEM((ROWS_PER_TILE,), jnp.float32))
    scratch.append(pltpu.VMEM_SHARED((NPAD, D), jnp.float32))
    if with_den:
        scratch.append(pltpu.VMEM_SHARED((NPAD,), jnp.float32))
    nsem = 12 if with_den else 8
    scratch += [pltpu.SemaphoreType.DMA] * nsem
    return pl.kernel(
        functools.partial(_sc_edge_body, with_den),
        out_type=(acc_t, den_t) if with_den else acc_t,
        mesh=mesh,
        compiler_params=pltpu.CompilerParams(
            use_tc_tiling_on_sc=False, needs_layout_passes=False),
        scratch_types=scratch,
    )


_SC_EDGE_CACHE = {}


def _sc_edge(with_den, *args):
    if with_den not in _SC_EDGE_CACHE:
        _SC_EDGE_CACHE[with_den] = _build_sc_edge(with_den)
    return _SC_EDGE_CACHE[with_den](*args)


def kernel(x, edge_index, W1, a_src1, a_dst1, b1, W2, a_src2, a_dst2, b2,
           Wc, bc):
    npad_e = E_PAD - N_EDGES
    src = jnp.concatenate(
        [edge_index[0].astype(jnp.int32), jnp.zeros((npad_e,), jnp.int32)]
    ).reshape(NW, NCHUNK, CHUNK)
    dst = jnp.concatenate(
        [edge_index[1].astype(jnp.int32),
         jnp.full((npad_e,), N_NODES, jnp.int32)]
    ).reshape(NW, NCHUNK, CHUNK)

    zpad = jnp.zeros((NPAD - N_NODES,), jnp.float32)

    h1, as1, ad1 = pl.pallas_call(
        _tc_layer1,
        out_shape=(
            jax.ShapeDtypeStruct((N_NODES, D), jnp.float32),
            jax.ShapeDtypeStruct((N_NODES,), jnp.float32),
            jax.ShapeDtypeStruct((N_NODES,), jnp.float32),
        ),
    )(x, W1, a_src1.reshape(1, D), a_dst1.reshape(1, D))

    acc1, den1 = _sc_edge(True, src, dst,
                          jnp.concatenate([as1, zpad]),
                          jnp.concatenate([ad1, zpad]), h1)

    W2p = jnp.concatenate([W2, jnp.zeros((16, 8), jnp.float32)], axis=1)
    a2sp = jnp.concatenate([a_src2, jnp.zeros((8,), jnp.float32)])
    a2dp = jnp.concatenate([a_dst2, jnp.zeros((8,), jnp.float32)])

    h2, as2, ad2 = pl.pallas_call(
        _tc_layer2,
        out_shape=(
            jax.ShapeDtypeStruct((N_NODES, D), jnp.float32),
            jax.ShapeDtypeStruct((N_NODES,), jnp.float32),
            jax.ShapeDtypeStruct((N_NODES,), jnp.float32),
        ),
    )(acc1, den1, b1.reshape(1, D), W2p, a2sp.reshape(1, D),
      a2dp.reshape(1, D))

    acc2 = _sc_edge(False, src, dst,
                    jnp.concatenate([as2, zpad]),
                    jnp.concatenate([ad2, zpad]), h2)

    h_out, scores = pl.pallas_call(
        _tc_head,
        out_shape=(
            jax.ShapeDtypeStruct((N_NODES, 8), jnp.float32),
            jax.ShapeDtypeStruct((N_NODES, 1), jnp.float32),
        ),
    )(acc2, b2.reshape(1, 8), Wc, bc.reshape(1, 1))

    return (h_out, scores)
